# baseline (speedup 1.0000x reference)
import functools

import jax
import jax.numpy as jnp
from jax import lax
from jax.experimental import pallas as pl
from jax.experimental.pallas import tpu as pltpu
from jax.experimental.pallas import tpu_sc as plsc

_B = 16384
_DG = 64
_DM = 512
_NW = 32
_CH = 64
_K = 4
_BS = _B // _K


def _make_mlp_gather(nrows):
    rpw = nrows // _NW
    nch = rpw // _CH
    mesh = plsc.VectorSubcoreMesh(core_axis_name="c", subcore_axis_name="s")

    @functools.partial(
        pl.kernel,
        out_type=[
            jax.ShapeDtypeStruct((nrows, _DM), jnp.float32),
            jax.ShapeDtypeStruct((nrows, _DM), jnp.float32),
        ],
        mesh=mesh,
        scratch_types=[
            pltpu.VMEM((_CH,), jnp.int32),
            pltpu.VMEM((_CH,), jnp.int32),
            pltpu.VMEM((_CH, _DM), jnp.float32),
            pltpu.VMEM((_CH, _DM), jnp.float32),
            pltpu.SemaphoreType.DMA,
        ],
    )
    def k(user_h, item_h, um_h, im_h, out_um, out_im,
          uidx, iidx, um_v, im_v, sem):
        wid = lax.axis_index("s") * 2 + lax.axis_index("c")
        base = wid * rpw
        for c in range(nch):
            row0 = base + c * _CH
            pltpu.sync_copy(user_h.at[pl.ds(row0, _CH)], uidx)
            pltpu.sync_copy(item_h.at[pl.ds(row0, _CH)], iidx)
            d3 = pltpu.async_copy(um_h.at[uidx], um_v, sem)
            d4 = pltpu.async_copy(im_h.at[iidx], im_v, sem)
            d3.wait()
            d4.wait()
            pltpu.sync_copy(um_v, out_um.at[pl.ds(row0, _CH)])
            pltpu.sync_copy(im_v, out_im.at[pl.ds(row0, _CH)])

    return k


def _make_gmf_gather(nrows):
    rpw = nrows // _NW
    nch = rpw // _CH
    mesh = plsc.VectorSubcoreMesh(core_axis_name="c", subcore_axis_name="s")

    @functools.partial(
        pl.kernel,
        out_type=[
            jax.ShapeDtypeStruct((nrows, 128), jnp.float32),
            jax.ShapeDtypeStruct((nrows, 128), jnp.float32),
        ],
        mesh=mesh,
        scratch_types=[
            pltpu.VMEM((_CH,), jnp.int32),
            pltpu.VMEM((_CH,), jnp.int32),
            pltpu.VMEM((_CH, 128), jnp.float32),
            pltpu.VMEM((_CH, 128), jnp.float32),
            pltpu.SemaphoreType.DMA,
        ],
    )
    def k(user_h, item_h, gmf_h, order_h, out_ug, out_ig,
          uidx, iidx, ug_v, ig_v, sem):
        del order_h
        wid = lax.axis_index("s") * 2 + lax.axis_index("c")
        base = wid * rpw
        for c in range(nch):
            row0 = base + c * _CH
            pltpu.sync_copy(user_h.at[pl.ds(row0, _CH)], uidx)
            pltpu.sync_copy(item_h.at[pl.ds(row0, _CH)], iidx)
            d1 = pltpu.async_copy(gmf_h.at[uidx], ug_v, sem)
            d2 = pltpu.async_copy(gmf_h.at[iidx], ig_v, sem)
            d1.wait()
            d2.wait()
            pltpu.sync_copy(ug_v, out_ug.at[pl.ds(row0, _CH)])
            pltpu.sync_copy(ig_v, out_ig.at[pl.ds(row0, _CH)])

    return k


_BM = 1024
_PREC = lax.Precision.DEFAULT


def _tc_body(eg, ig, eum, eim, r, w0u, w0i, w1, w2, w3,
             bb0, bb1, bb2, bb3, wg, wm, wt, out):
    x0 = (jnp.dot(eum[...], w0u[...], preferred_element_type=jnp.float32,
                  precision=_PREC)
          + jnp.dot(eim[...], w0i[...], preferred_element_type=jnp.float32,
                    precision=_PREC)
          + bb0[...])
    h = jnp.maximum(x0, 0.0)
    h = jnp.maximum(jnp.dot(h, w1[...], preferred_element_type=jnp.float32,
                            precision=_PREC) + bb1[...], 0.0)
    h = jnp.maximum(jnp.dot(h, w2[...], preferred_element_type=jnp.float32,
                            precision=_PREC) + bb2[...], 0.0)
    h = jnp.maximum(jnp.dot(h, w3[...], preferred_element_type=jnp.float32,
                            precision=_PREC) + bb3[...], 0.0)
    gmf = eg[...][:, :_DG] * ig[...][:, _DG:]
    s = jnp.sum(gmf * wg[...], axis=1) + jnp.sum(h * wm[...], axis=1)
    out[0, 0, :] = s + r[0, 0, :] * wt[0, 0] + wt[0, 1]


def _tc_mlp(eg, ig, eum, eim, r2, w0u, w0i, w1, w2, w3,
            b0, b1, b2, b3, wg, wm, wt):
    nb = eg.shape[0] // _BM
    full = lambda a: pl.BlockSpec(a.shape, lambda i: (0, 0))
    blk = lambda n: pl.BlockSpec((_BM, n), lambda i: (i, 0))
    row = pl.BlockSpec((1, 1, _BM), lambda i: (i, 0, 0))
    return pl.pallas_call(
        _tc_body,
        grid=(nb,),
        in_specs=[
            blk(128), blk(128), blk(_DM), blk(_DM), row,
            full(w0u), full(w0i), full(w1), full(w2), full(w3),
            full(b0), full(b1), full(b2), full(b3),
            full(wg), full(wm), full(wt),
        ],
        out_specs=row,
        out_shape=jax.ShapeDtypeStruct((nb, 1, _BM), jnp.float32),
    )(eg, ig, eum, eim, r2, w0u, w0i, w1, w2, w3,
      b0, b1, b2, b3, wg, wm, wt)


def kernel(user, item, rating, emb_user_gmf, emb_item_gmf,
           emb_user_mlp, emb_item_mlp,
           W0, b0, W1, b1, W2, b2, W3, b3, Wp, bp):
    user = user.astype(jnp.int32)
    item = item.astype(jnp.int32)
    gmf_cat = jnp.concatenate([emb_user_gmf, emb_item_gmf], axis=1)

    w0t = W0.T
    wargs = (
        w0t[:_DM], w0t[_DM:], W1.T, W2.T, W3.T,
        b0[None, :], b1[None, :], b2[None, :], b3[None, :],
        Wp[:, :_DG], Wp[:, _DG:2 * _DG],
        jnp.concatenate([Wp[:, 2 * _DG:], bp[None, :]], axis=1),
    )

    mlp_g = _make_mlp_gather(_BS)
    gmf_g = _make_gmf_gather(_BS)
    us = [user[i * _BS:(i + 1) * _BS] for i in range(_K)]
    its = [item[i * _BS:(i + 1) * _BS] for i in range(_K)]
    mlp_rows = [mlp_g(us[i], its[i], emb_user_mlp, emb_item_mlp)
                for i in range(_K)]
    gmf_rows = [gmf_g(us[i], its[i], gmf_cat, mlp_rows[-1][0])
                for i in range(_K)]
    outs = [
        _tc_mlp(gmf_rows[i][0], gmf_rows[i][1],
                mlp_rows[i][0], mlp_rows[i][1],
                rating[i * _BS:(i + 1) * _BS].reshape(-1, 1, _BM), *wargs)
        for i in range(_K)
    ]
    return jnp.concatenate([o.reshape(-1) for o in outs])


# speedup vs baseline: 1.1061x; 1.1061x over previous
import functools

import jax
import jax.numpy as jnp
from jax import lax
from jax.experimental import pallas as pl
from jax.experimental.pallas import tpu as pltpu
from jax.experimental.pallas import tpu_sc as plsc

_B = 16384
_DG = 64
_DM = 512
_NW = 32
_CH = 64
_K = 4
_BS = _B // _K


def _make_sc_gather(nrows):
    rpw = nrows // _NW
    nch = rpw // _CH
    mesh = plsc.VectorSubcoreMesh(core_axis_name="c", subcore_axis_name="s")

    @functools.partial(
        pl.kernel,
        out_type=[
            jax.ShapeDtypeStruct((nrows, 128), jnp.float32),
            jax.ShapeDtypeStruct((nrows, 128), jnp.float32),
            jax.ShapeDtypeStruct((nrows, _DM), jnp.float32),
            jax.ShapeDtypeStruct((nrows, _DM), jnp.float32),
        ],
        mesh=mesh,
        scratch_types=[
            pltpu.VMEM((_CH,), jnp.int32),
            pltpu.VMEM((_CH,), jnp.int32),
            pltpu.VMEM((_CH, 128), jnp.float32),
            pltpu.VMEM((_CH, 128), jnp.float32),
            pltpu.VMEM((_CH, _DM), jnp.float32),
            pltpu.VMEM((_CH, _DM), jnp.float32),
            pltpu.SemaphoreType.DMA,
        ],
    )
    def k(user_h, item_h, gmf_h, um_h, im_h,
          out_ug, out_ig, out_um, out_im,
          uidx, iidx, ug_v, ig_v, um_v, im_v, sem):
        wid = lax.axis_index("s") * 2 + lax.axis_index("c")
        base = wid * rpw
        for c in range(nch):
            row0 = base + c * _CH
            pltpu.sync_copy(user_h.at[pl.ds(row0, _CH)], uidx)
            pltpu.sync_copy(item_h.at[pl.ds(row0, _CH)], iidx)
            d1 = pltpu.async_copy(gmf_h.at[uidx], ug_v, sem)
            d2 = pltpu.async_copy(gmf_h.at[iidx], ig_v, sem)
            d3 = pltpu.async_copy(um_h.at[uidx], um_v, sem)
            d4 = pltpu.async_copy(im_h.at[iidx], im_v, sem)
            d1.wait()
            d2.wait()
            d3.wait()
            d4.wait()
            pltpu.sync_copy(ug_v, out_ug.at[pl.ds(row0, _CH)])
            pltpu.sync_copy(ig_v, out_ig.at[pl.ds(row0, _CH)])
            pltpu.sync_copy(um_v, out_um.at[pl.ds(row0, _CH)])
            pltpu.sync_copy(im_v, out_im.at[pl.ds(row0, _CH)])

    return k


_BM = 1024
_PREC = lax.Precision.DEFAULT


def _tc_body(eg, ig, eum, eim, r, w0u, w0i, w1, w2, w3,
             bb0, bb1, bb2, bb3, wg, wm, wt, out):
    x0 = (jnp.dot(eum[...], w0u[...], preferred_element_type=jnp.float32,
                  precision=_PREC)
          + jnp.dot(eim[...], w0i[...], preferred_element_type=jnp.float32,
                    precision=_PREC)
          + bb0[...])
    h = jnp.maximum(x0, 0.0)
    h = jnp.maximum(jnp.dot(h, w1[...], preferred_element_type=jnp.float32,
                            precision=_PREC) + bb1[...], 0.0)
    h = jnp.maximum(jnp.dot(h, w2[...], preferred_element_type=jnp.float32,
                            precision=_PREC) + bb2[...], 0.0)
    h = jnp.maximum(jnp.dot(h, w3[...], preferred_element_type=jnp.float32,
                            precision=_PREC) + bb3[...], 0.0)
    gmf = eg[...][:, :_DG] * ig[...][:, _DG:]
    s = jnp.sum(gmf * wg[...], axis=1) + jnp.sum(h * wm[...], axis=1)
    out[0, 0, :] = s + r[0, 0, :] * wt[0, 0] + wt[0, 1]


def _tc_mlp(eg, ig, eum, eim, r2, w0u, w0i, w1, w2, w3,
            b0, b1, b2, b3, wg, wm, wt):
    nb = eg.shape[0] // _BM
    full = lambda a: pl.BlockSpec(a.shape, lambda i: (0, 0))
    blk = lambda n: pl.BlockSpec((_BM, n), lambda i: (i, 0))
    row = pl.BlockSpec((1, 1, _BM), lambda i: (i, 0, 0))
    return pl.pallas_call(
        _tc_body,
        grid=(nb,),
        in_specs=[
            blk(128), blk(128), blk(_DM), blk(_DM), row,
            full(w0u), full(w0i), full(w1), full(w2), full(w3),
            full(b0), full(b1), full(b2), full(b3),
            full(wg), full(wm), full(wt),
        ],
        out_specs=row,
        out_shape=jax.ShapeDtypeStruct((nb, 1, _BM), jnp.float32),
    )(eg, ig, eum, eim, r2, w0u, w0i, w1, w2, w3,
      b0, b1, b2, b3, wg, wm, wt)


def kernel(user, item, rating, emb_user_gmf, emb_item_gmf,
           emb_user_mlp, emb_item_mlp,
           W0, b0, W1, b1, W2, b2, W3, b3, Wp, bp):
    user = user.astype(jnp.int32)
    item = item.astype(jnp.int32)
    gmf_cat = jnp.concatenate([emb_user_gmf, emb_item_gmf], axis=1)

    w0t = W0.T
    wargs = (
        w0t[:_DM], w0t[_DM:], W1.T, W2.T, W3.T,
        b0[None, :], b1[None, :], b2[None, :], b3[None, :],
        Wp[:, :_DG], Wp[:, _DG:2 * _DG],
        jnp.concatenate([Wp[:, 2 * _DG:], bp[None, :]], axis=1),
    )

    sc = _make_sc_gather(_BS)
    gathered = [
        sc(user[i * _BS:(i + 1) * _BS], item[i * _BS:(i + 1) * _BS],
           gmf_cat, emb_user_mlp, emb_item_mlp)
        for i in range(_K)
    ]
    outs = [
        _tc_mlp(*g, rating[i * _BS:(i + 1) * _BS].reshape(-1, 1, _BM), *wargs)
        for i, g in enumerate(gathered)
    ]
    return jnp.concatenate([o.reshape(-1) for o in outs])
